# baseline (speedup 1.0000x reference)
import functools

import jax
import jax.numpy as jnp
from jax import lax
from jax.experimental import pallas as pl
from jax.experimental.pallas import tpu as pltpu
from jax.experimental.pallas import tpu_sc as plsc

_N = 10000
_E = 320000
_H = 128
_OUT = 10
_NPAD = 10240
_NC = 2
_NS = 16
_NW = _NC * _NS
_C = 128
_CPT = 80
_CPH = _CPT // 2
_EPAD = _C * _CPT * _NW
_ROWS_PER_TILE = _NPAD // _NS
_KS = (5000, 2500, 1250)
_INT_MIN_PY = -(2 ** 31)


_NSLOT = 2


def _seg_sum_body(x_hbm, src_hbm, dst_hbm, zrow_hbm, out_hbm,
                  sidx, didx, rows0, rows1, agg, gsem, ssem):
    c = lax.axis_index("c")
    s = lax.axis_index("s")
    w = c * _NS + s
    rows = (rows0, rows1)

    pltpu.sync_copy(zrow_hbm, rows0)
    for i in range(_ROWS_PER_TILE // _C):
        pltpu.sync_copy(rows0, agg.at[pl.ds(s * _ROWS_PER_TILE + i * _C, _C)])
    plsc.subcore_barrier()

    for h in range(_CPT // _CPH):
        pltpu.sync_copy(src_hbm.at[pl.ds(w * _CPT + h * _CPH, _CPH)], sidx)
        pltpu.sync_copy(dst_hbm.at[pl.ds(w * _CPT + h * _CPH, _CPH)], didx)
        for b in range(_NSLOT):
            pltpu.async_copy(x_hbm.at[sidx.at[b]], rows[b], gsem[b])

        def ebody(jj, carry):
            for b in range(_NSLOT):
                cur = jj * _NSLOT + b
                pltpu.make_async_copy(x_hbm.at[sidx.at[cur]], rows[b],
                                      gsem[b]).wait()
                pltpu.async_copy(rows[b], agg.at[didx.at[cur]], ssem[b],
                                 add=True)
                nxt = cur + _NSLOT

                @pl.when(nxt < _CPH)
                def _():
                    pltpu.make_async_copy(rows[b], agg.at[didx.at[cur]],
                                          ssem[b]).wait()
                    pltpu.async_copy(x_hbm.at[sidx.at[nxt]], rows[b], gsem[b])
            return carry

        lax.fori_loop(0, _CPH // _NSLOT, ebody, 0)
        for b in range(_NSLOT):
            pltpu.make_async_copy(rows[b], agg.at[didx.at[0]], ssem[b]).wait()
    plsc.subcore_barrier()

    pltpu.sync_copy(agg.at[pl.ds(s * _ROWS_PER_TILE, _ROWS_PER_TILE)],
                    out_hbm.at[c, pl.ds(s * _ROWS_PER_TILE, _ROWS_PER_TILE)])


def _seg_sum_sc(x_pad, src2d, dst2d, zrow):
    mesh = plsc.VectorSubcoreMesh(core_axis_name="c", subcore_axis_name="s",
                                  num_cores=_NC, num_subcores=_NS)
    return pl.kernel(
        _seg_sum_body,
        out_type=jax.ShapeDtypeStruct((_NC, _NPAD, _H), jnp.float32),
        mesh=mesh,
        scratch_types=[
            pltpu.VMEM((_CPH, _C), jnp.int32),
            pltpu.VMEM((_CPH, _C), jnp.int32),
            pltpu.VMEM((_C, _H), jnp.float32),
            pltpu.VMEM((_C, _H), jnp.float32),
            pltpu.VMEM_SHARED((_NPAD, _H), jnp.float32),
            [pltpu.SemaphoreType.DMA] * _NSLOT,
            [pltpu.SemaphoreType.DMA] * _NSLOT,
        ],
    )(x_pad, src2d, dst2d, zrow)


def _orderable_key(score):
    raw = lax.bitcast_convert_type(score, jnp.int32)
    return jnp.where(raw >= 0, raw, jnp.int32(_INT_MIN_PY) - raw)


def _count(pred):
    return jnp.sum(pred.astype(jnp.int32))


def _kth_largest(keys, mask, t):
    int_min = jnp.int32(_INT_MIN_PY)
    c_pos = _count(mask & (keys >= 0))
    use_pos = c_pos >= t
    tt = jnp.where(use_pos, t, t - c_pos)
    tkeys = jnp.where(use_pos, keys, keys ^ int_min)

    def body(i, prefix):
        cand = prefix | lax.shift_left(jnp.int32(1), 30 - i)
        cnt = _count(mask & (tkeys >= cand))
        return jnp.where(cnt >= tt, cand, prefix)

    v = lax.fori_loop(0, 31, body, jnp.int32(0))
    return jnp.where(use_pos, v, v ^ int_min)


def _tth_smallest_nonneg(keys, mask, t, nbits):
    def body(i, prefix):
        cand = prefix | lax.shift_left(jnp.int32(1), nbits - 1 - i)
        cnt = _count(mask & (keys < cand))
        return jnp.where(cnt >= t, prefix, cand)

    return lax.fori_loop(0, nbits, body, jnp.int32(0))


_RB = 1024
_NB = _NPAD // _RB
_SR = _NPAD // 128


def _lane_mask():
    return (lax.broadcasted_iota(jnp.int32, (_NPAD, 128), 1)
            == lax.broadcasted_iota(jnp.int32, (_NPAD, 128), 0) % 128
            ).astype(jnp.float32)


def _col_to_grid(col):
    brep_t = (lax.broadcasted_iota(jnp.int32, (_SR, _NPAD), 1) // 128
              == lax.broadcasted_iota(jnp.int32, (_SR, _NPAD), 0)
              ).astype(jnp.float32)
    return jnp.dot(brep_t, col * _lane_mask(),
                   precision=lax.Precision.HIGHEST)


def _grid_to_col(x80):
    brep = (lax.broadcasted_iota(jnp.int32, (_NPAD, _SR), 0) // 128
            == lax.broadcasted_iota(jnp.int32, (_NPAD, _SR), 1)
            ).astype(jnp.float32)
    expanded = jnp.dot(brep, x80, precision=lax.Precision.HIGHEST)
    return jnp.sum(expanded * _lane_mask(), axis=1, keepdims=True)


def _fused_layer_body(k_sel, n_prev, *refs):
    (p_ref, x_ref, m_ref, wrel_ref, wroot_ref, brel_ref, pw_ref) = refs[:7]
    prev_key_refs = refs[7:7 + n_prev]
    (xo_ref, mo_ref, g_ref, key_ref) = refs[7 + n_prev:]

    agg = p_ref[0] + p_ref[1]
    h = jnp.maximum(
        jnp.dot(agg, wrel_ref[...]) + brel_ref[...]
        + jnp.dot(x_ref[...], wroot_ref[...]), 0.0)
    pw = pw_ref[...]
    s_col = jnp.tanh(jnp.dot(h, pw) / jnp.sqrt(jnp.sum(pw * pw)))
    score = _col_to_grid(s_col)
    act = m_ref[...] > 0
    skey = _orderable_key(score)

    v = _kth_largest(skey, act, k_sel)
    sel = act & (skey > v)
    t = k_sel - _count(sel)
    tie = act & (skey == v)
    for pk_ref in prev_key_refs:
        pk = pk_ref[...]
        u = _kth_largest(pk, tie, t)
        g2 = tie & (pk > u)
        sel = sel | g2
        t = t - _count(g2)
        tie = tie & (pk == u)
    idx = (lax.broadcasted_iota(jnp.int32, (_SR, 128), 0) * 128
           + lax.broadcasted_iota(jnp.int32, (_SR, 128), 1))
    cut = _tth_smallest_nonneg(idx, tie, t, 14)
    sel = sel | (tie & (idx <= cut))
    mnew = sel.astype(jnp.float32)

    mo_ref[...] = mnew
    key_ref[...] = skey
    m_col = _grid_to_col(mnew)
    xo = h * (s_col * m_col)
    xo_ref[...] = xo
    gsum = jnp.sum(xo, axis=0, keepdims=True)
    gmax = jnp.max(jnp.where(m_col > 0, xo, -jnp.inf), axis=0, keepdims=True)
    g_ref[...] = jnp.broadcast_to(
        jnp.concatenate([gsum, gmax], axis=1), (8, 2 * _H))


def _fused_layer_tc(p, xcur, m80, prev_keys, Wrel, Wroot, brel2d, pw2d, k_sel):
    return pl.pallas_call(
        functools.partial(_fused_layer_body, k_sel, len(prev_keys)),
        out_shape=[
            jax.ShapeDtypeStruct((_NPAD, _H), jnp.float32),
            jax.ShapeDtypeStruct((_SR, 128), jnp.float32),
            jax.ShapeDtypeStruct((8, 2 * _H), jnp.float32),
            jax.ShapeDtypeStruct((_SR, 128), jnp.int32),
        ],
    )(p, xcur, m80, Wrel, Wroot, brel2d, pw2d, *prev_keys)


def _conv_body(p_ref, x_ref, wrel_ref, wroot_ref, brel_ref, pw_ref,
               h_ref, s_ref):
    agg = p_ref[0] + p_ref[1]
    h = jnp.maximum(
        jnp.dot(agg, wrel_ref[...]) + brel_ref[...]
        + jnp.dot(x_ref[...], wroot_ref[...]), 0.0)
    h_ref[...] = h
    pw = pw_ref[...]
    s_ref[...] = jnp.tanh(jnp.dot(h, pw) / jnp.sqrt(jnp.sum(pw * pw)))


def _conv_tc(p, xcur, Wrel, Wroot, brel2d, pw2d):
    return pl.pallas_call(
        _conv_body,
        grid=(_NB,),
        in_specs=[
            pl.BlockSpec((2, _RB, _H), lambda i: (0, i, 0)),
            pl.BlockSpec((_RB, _H), lambda i: (i, 0)),
            pl.BlockSpec((_H, _H), lambda i: (0, 0)),
            pl.BlockSpec((_H, _H), lambda i: (0, 0)),
            pl.BlockSpec((1, _H), lambda i: (0, 0)),
            pl.BlockSpec((_H, 1), lambda i: (0, 0)),
        ],
        out_specs=[
            pl.BlockSpec((_RB, _H), lambda i: (i, 0)),
            pl.BlockSpec((_RB, 1), lambda i: (i, 0)),
        ],
        out_shape=[
            jax.ShapeDtypeStruct((_NPAD, _H), jnp.float32),
            jax.ShapeDtypeStruct((_NPAD, 1), jnp.float32),
        ],
    )(p, xcur, Wrel, Wroot, brel2d, pw2d)


def _select_body(k_sel, n_prev, *refs):
    s_ref, m_ref = refs[0], refs[1]
    prev_key_refs = refs[2:2 + n_prev]
    mo_ref, smo_ref, key_ref = refs[2 + n_prev:]

    score = s_ref[...]
    act = m_ref[...] > 0
    skey = _orderable_key(score)

    v = _kth_largest(skey, act, k_sel)
    sel = act & (skey > v)
    t = k_sel - _count(sel)
    tie = act & (skey == v)
    for pk_ref in prev_key_refs:
        pk = pk_ref[...]
        u = _kth_largest(pk, tie, t)
        g2 = tie & (pk > u)
        sel = sel | g2
        t = t - _count(g2)
        tie = tie & (pk == u)
    idx = (lax.broadcasted_iota(jnp.int32, (_SR, 128), 0) * 128
           + lax.broadcasted_iota(jnp.int32, (_SR, 128), 1))
    cut = _tth_smallest_nonneg(idx, tie, t, 14)
    sel = sel | (tie & (idx <= cut))
    mnew = sel.astype(jnp.float32)

    mo_ref[...] = mnew
    smo_ref[...] = score * mnew
    key_ref[...] = skey


def _select_tc(s80, m80, prev_keys, k_sel):
    return pl.pallas_call(
        functools.partial(_select_body, k_sel, len(prev_keys)),
        out_shape=[
            jax.ShapeDtypeStruct((_SR, 128), jnp.float32),
            jax.ShapeDtypeStruct((_SR, 128), jnp.float32),
            jax.ShapeDtypeStruct((_SR, 128), jnp.int32),
        ],
    )(s80, m80, *prev_keys)


def _scale_body(h_ref, sm_ref, mc_ref, xo_ref, g_ref):
    i = pl.program_id(0)
    xo = h_ref[...] * sm_ref[...]
    xo_ref[...] = xo
    bsum = jnp.sum(xo, axis=0, keepdims=True)
    bmax = jnp.max(jnp.where(mc_ref[...] > 0, xo, -jnp.inf),
                   axis=0, keepdims=True)
    cur = jnp.concatenate([bsum, bmax], axis=1)

    @pl.when(i == 0)
    def _():
        g_ref[...] = jnp.broadcast_to(cur, (8, 2 * _H))

    @pl.when(i != 0)
    def _():
        prev = g_ref[0:1, :]
        comb = jnp.concatenate(
            [prev[:, :_H] + bsum, jnp.maximum(prev[:, _H:], bmax)], axis=1)
        g_ref[...] = jnp.broadcast_to(comb, (8, 2 * _H))


def _scale_tc(h, sm_col, m_col):
    return pl.pallas_call(
        _scale_body,
        grid=(_NB,),
        in_specs=[
            pl.BlockSpec((_RB, _H), lambda i: (i, 0)),
            pl.BlockSpec((_RB, 1), lambda i: (i, 0)),
            pl.BlockSpec((_RB, 1), lambda i: (i, 0)),
        ],
        out_specs=[
            pl.BlockSpec((_RB, _H), lambda i: (i, 0)),
            pl.BlockSpec((8, 2 * _H), lambda i: (0, 0)),
        ],
        out_shape=[
            jax.ShapeDtypeStruct((_NPAD, _H), jnp.float32),
            jax.ShapeDtypeStruct((8, 2 * _H), jnp.float32),
        ],
    )(h, sm_col, m_col)


def _final_body(g0_ref, g1_ref, g2_ref, w1_ref, b1_ref, w2_ref, b2_ref, o_ref):
    gs = [g0_ref[0:1, :], g1_ref[0:1, :], g2_ref[0:1, :]]
    gmean = sum(g[:, :_H] / _KS[i] for i, g in enumerate(gs))
    gmax = gs[0][:, _H:] + gs[1][:, _H:] + gs[2][:, _H:]
    g = jnp.concatenate([gmean, gmax], axis=1)
    h = jnp.maximum(jnp.dot(g, w1_ref[...]) + b1_ref[...], 0.0)
    o = jnp.dot(h, w2_ref[...]) + b2_ref[...]
    o_ref[...] = jnp.broadcast_to(o, (8, _H))


def _final_tc(g0, g1, g2, W1, b1, W2p, b2p):
    return pl.pallas_call(
        _final_body,
        out_shape=jax.ShapeDtypeStruct((8, _H), jnp.float32),
    )(g0, g1, g2, W1, b1, W2p, b2p)


def kernel(x, edge_index, batch,
           Wrel0, brel0, Wroot0, pw0,
           Wrel1, brel1, Wroot1, pw1,
           Wrel2, brel2, Wroot2, pw2,
           lin1_W, lin1_b, lin2_W, lin2_b):
    f32 = jnp.float32
    src = edge_index[0].astype(jnp.int32)
    dst = edge_index[1].astype(jnp.int32)
    pad = _N + (jnp.arange(_EPAD - _E, dtype=jnp.int32) % (_NPAD - _N))
    src2d = jnp.concatenate([src, pad]).reshape(_EPAD // _C, _C)
    dst2d = jnp.concatenate([dst, pad]).reshape(_EPAD // _C, _C)
    xp = jnp.zeros((_NPAD, _H), f32).at[:_N].set(x.astype(f32))
    zrow = jnp.zeros((_C, _H), f32)
    m80 = (jnp.arange(_NPAD, dtype=jnp.int32) < _N).astype(f32).reshape(_SR, 128)
    W2p = jnp.zeros((_H, _H), f32).at[:, :_OUT].set(lin2_W)
    b2p = jnp.zeros((1, _H), f32).at[0, :_OUT].set(lin2_b)

    params = ((Wrel0, brel0, Wroot0, pw0),
              (Wrel1, brel1, Wroot1, pw1),
              (Wrel2, brel2, Wroot2, pw2))
    gs = []
    prev_keys = []
    for i in range(3):
        Wrel, brel, Wroot, pw = params[i]
        p = _seg_sum_sc(xp, src2d, dst2d, zrow)
        h, s_col = _conv_tc(p, xp, Wrel, Wroot,
                            brel.reshape(1, _H), pw.reshape(_H, 1))
        m80, sm80, key80 = _select_tc(s_col.reshape(_SR, 128), m80,
                                      prev_keys, _KS[i])
        xp, g = _scale_tc(h, sm80.reshape(_NPAD, 1), m80.reshape(_NPAD, 1))
        gs.append(g)
        prev_keys.insert(0, key80)
    o8 = _final_tc(gs[0], gs[1], gs[2], lin1_W, lin1_b.reshape(1, _H),
                   W2p, b2p)
    return (o8[0:1, :_OUT], jnp.zeros(()))


# speedup vs baseline: 1.0626x; 1.0626x over previous
import functools

import jax
import jax.numpy as jnp
from jax import lax
from jax.experimental import pallas as pl
from jax.experimental.pallas import tpu as pltpu
from jax.experimental.pallas import tpu_sc as plsc

_N = 10000
_E = 320000
_H = 128
_OUT = 10
_NPAD = 10240
_NC = 2
_NS = 16
_NW = _NC * _NS
_C = 64
_CPT = 160
_CPH = _CPT // 4
_EPAD = _C * _CPT * _NW
_ROWS_PER_TILE = _NPAD // _NS
_KS = (5000, 2500, 1250)
_INT_MIN_PY = -(2 ** 31)


_NSLOT = 4
_ZR = 32


def _seg_sum_body(x_hbm, src_hbm, dst_hbm, zrow_hbm, out_hbm,
                  sidx, didx, rows0, rows1, rows2, rows3, zv, agg,
                  gsem, ssem, zsem):
    c = lax.axis_index("c")
    s = lax.axis_index("s")
    w = c * _NS + s
    rows = (rows0, rows1, rows2, rows3)
    nzero = _ROWS_PER_TILE // _ZR

    pltpu.sync_copy(zrow_hbm, zv)
    for i in range(nzero):
        pltpu.async_copy(zv, agg.at[pl.ds(s * _ROWS_PER_TILE + i * _ZR, _ZR)],
                         zsem)

    for h in range(_CPT // _CPH):
        pltpu.sync_copy(src_hbm.at[pl.ds(w * _CPT + h * _CPH, _CPH)], sidx)
        pltpu.sync_copy(dst_hbm.at[pl.ds(w * _CPT + h * _CPH, _CPH)], didx)
        for b in range(_NSLOT):
            pltpu.async_copy(x_hbm.at[sidx.at[b]], rows[b], gsem[b])
        if h == 0:
            for i in range(nzero):
                pltpu.make_async_copy(
                    zv, agg.at[pl.ds(s * _ROWS_PER_TILE + i * _ZR, _ZR)],
                    zsem).wait()
            plsc.subcore_barrier()

        def ebody(jj, carry):
            for b in range(_NSLOT):
                cur = jj * _NSLOT + b
                pltpu.make_async_copy(x_hbm.at[sidx.at[cur]], rows[b],
                                      gsem[b]).wait()
                pltpu.async_copy(rows[b], agg.at[didx.at[cur]], ssem[b],
                                 add=True)
                nxt = cur + _NSLOT

                @pl.when(nxt < _CPH)
                def _():
                    pltpu.make_async_copy(rows[b], agg.at[didx.at[cur]],
                                          ssem[b]).wait()
                    pltpu.async_copy(x_hbm.at[sidx.at[nxt]], rows[b], gsem[b])
            return carry

        lax.fori_loop(0, _CPH // _NSLOT, ebody, 0)
        for b in range(_NSLOT):
            pltpu.make_async_copy(rows[b], agg.at[didx.at[0]], ssem[b]).wait()
    plsc.subcore_barrier()

    pltpu.sync_copy(agg.at[pl.ds(s * _ROWS_PER_TILE, _ROWS_PER_TILE)],
                    out_hbm.at[c, pl.ds(s * _ROWS_PER_TILE, _ROWS_PER_TILE)])


def _seg_sum_sc(x_pad, src2d, dst2d, zrow):
    mesh = plsc.VectorSubcoreMesh(core_axis_name="c", subcore_axis_name="s",
                                  num_cores=_NC, num_subcores=_NS)
    return pl.kernel(
        _seg_sum_body,
        out_type=jax.ShapeDtypeStruct((_NC, _NPAD, _H), jnp.float32),
        mesh=mesh,
        scratch_types=[
            pltpu.VMEM((_CPH, _C), jnp.int32),
            pltpu.VMEM((_CPH, _C), jnp.int32),
            pltpu.VMEM((_C, _H), jnp.float32),
            pltpu.VMEM((_C, _H), jnp.float32),
            pltpu.VMEM((_C, _H), jnp.float32),
            pltpu.VMEM((_C, _H), jnp.float32),
            pltpu.VMEM((_ZR, _H), jnp.float32),
            pltpu.VMEM_SHARED((_NPAD, _H), jnp.float32),
            [pltpu.SemaphoreType.DMA] * _NSLOT,
            [pltpu.SemaphoreType.DMA] * _NSLOT,
            pltpu.SemaphoreType.DMA,
        ],
    )(x_pad, src2d, dst2d, zrow)


def _orderable_key(score):
    raw = lax.bitcast_convert_type(score, jnp.int32)
    return jnp.where(raw >= 0, raw, jnp.int32(_INT_MIN_PY) - raw)


def _count(pred):
    return jnp.sum(pred.astype(jnp.int32))


def _kth_largest(keys, mask, t):
    int_min = jnp.int32(_INT_MIN_PY)
    c_pos = _count(mask & (keys >= 0))
    use_pos = c_pos >= t
    tt = jnp.where(use_pos, t, t - c_pos)
    tkeys = jnp.where(use_pos, keys, keys ^ int_min)

    def body(i, prefix):
        cand = prefix | lax.shift_left(jnp.int32(1), 30 - i)
        cnt = _count(mask & (tkeys >= cand))
        return jnp.where(cnt >= tt, cand, prefix)

    v = lax.fori_loop(0, 31, body, jnp.int32(0))
    return jnp.where(use_pos, v, v ^ int_min)


def _tth_smallest_nonneg(keys, mask, t, nbits):
    def body(i, prefix):
        cand = prefix | lax.shift_left(jnp.int32(1), nbits - 1 - i)
        cnt = _count(mask & (keys < cand))
        return jnp.where(cnt >= t, prefix, cand)

    return lax.fori_loop(0, nbits, body, jnp.int32(0))


_RB = 1024
_NB = _NPAD // _RB
_SR = _NPAD // 128


def _lane_mask():
    return (lax.broadcasted_iota(jnp.int32, (_NPAD, 128), 1)
            == lax.broadcasted_iota(jnp.int32, (_NPAD, 128), 0) % 128
            ).astype(jnp.float32)


def _col_to_grid(col):
    brep_t = (lax.broadcasted_iota(jnp.int32, (_SR, _NPAD), 1) // 128
              == lax.broadcasted_iota(jnp.int32, (_SR, _NPAD), 0)
              ).astype(jnp.float32)
    return jnp.dot(brep_t, col * _lane_mask(),
                   precision=lax.Precision.HIGHEST)


def _grid_to_col(x80):
    brep = (lax.broadcasted_iota(jnp.int32, (_NPAD, _SR), 0) // 128
            == lax.broadcasted_iota(jnp.int32, (_NPAD, _SR), 1)
            ).astype(jnp.float32)
    expanded = jnp.dot(brep, x80, precision=lax.Precision.HIGHEST)
    return jnp.sum(expanded * _lane_mask(), axis=1, keepdims=True)


def _fused_layer_body(k_sel, n_prev, *refs):
    (p_ref, x_ref, m_ref, wrel_ref, wroot_ref, brel_ref, pw_ref) = refs[:7]
    prev_key_refs = refs[7:7 + n_prev]
    (xo_ref, mo_ref, g_ref, key_ref) = refs[7 + n_prev:]

    agg = p_ref[0] + p_ref[1]
    h = jnp.maximum(
        jnp.dot(agg, wrel_ref[...]) + brel_ref[...]
        + jnp.dot(x_ref[...], wroot_ref[...]), 0.0)
    pw = pw_ref[...]
    s_col = jnp.tanh(jnp.dot(h, pw) / jnp.sqrt(jnp.sum(pw * pw)))
    score = _col_to_grid(s_col)
    act = m_ref[...] > 0
    skey = _orderable_key(score)

    v = _kth_largest(skey, act, k_sel)
    sel = act & (skey > v)
    t = k_sel - _count(sel)
    tie = act & (skey == v)
    for pk_ref in prev_key_refs:
        pk = pk_ref[...]
        u = _kth_largest(pk, tie, t)
        g2 = tie & (pk > u)
        sel = sel | g2
        t = t - _count(g2)
        tie = tie & (pk == u)
    idx = (lax.broadcasted_iota(jnp.int32, (_SR, 128), 0) * 128
           + lax.broadcasted_iota(jnp.int32, (_SR, 128), 1))
    cut = _tth_smallest_nonneg(idx, tie, t, 14)
    sel = sel | (tie & (idx <= cut))
    mnew = sel.astype(jnp.float32)

    mo_ref[...] = mnew
    key_ref[...] = skey
    m_col = _grid_to_col(mnew)
    xo = h * (s_col * m_col)
    xo_ref[...] = xo
    gsum = jnp.sum(xo, axis=0, keepdims=True)
    gmax = jnp.max(jnp.where(m_col > 0, xo, -jnp.inf), axis=0, keepdims=True)
    g_ref[...] = jnp.broadcast_to(
        jnp.concatenate([gsum, gmax], axis=1), (8, 2 * _H))


def _fused_layer_tc(p, xcur, m80, prev_keys, Wrel, Wroot, brel2d, pw2d, k_sel):
    return pl.pallas_call(
        functools.partial(_fused_layer_body, k_sel, len(prev_keys)),
        out_shape=[
            jax.ShapeDtypeStruct((_NPAD, _H), jnp.float32),
            jax.ShapeDtypeStruct((_SR, 128), jnp.float32),
            jax.ShapeDtypeStruct((8, 2 * _H), jnp.float32),
            jax.ShapeDtypeStruct((_SR, 128), jnp.int32),
        ],
    )(p, xcur, m80, Wrel, Wroot, brel2d, pw2d, *prev_keys)


def _conv_body(p_ref, x_ref, wrel_ref, wroot_ref, brel_ref, pw_ref,
               h_ref, s_ref):
    agg = p_ref[0] + p_ref[1]
    h = jnp.maximum(
        jnp.dot(agg, wrel_ref[...]) + brel_ref[...]
        + jnp.dot(x_ref[...], wroot_ref[...]), 0.0)
    h_ref[...] = h
    pw = pw_ref[...]
    s_ref[...] = jnp.tanh(jnp.dot(h, pw) / jnp.sqrt(jnp.sum(pw * pw)))


def _conv_tc(p, xcur, Wrel, Wroot, brel2d, pw2d):
    return pl.pallas_call(
        _conv_body,
        grid=(_NB,),
        in_specs=[
            pl.BlockSpec((2, _RB, _H), lambda i: (0, i, 0)),
            pl.BlockSpec((_RB, _H), lambda i: (i, 0)),
            pl.BlockSpec((_H, _H), lambda i: (0, 0)),
            pl.BlockSpec((_H, _H), lambda i: (0, 0)),
            pl.BlockSpec((1, _H), lambda i: (0, 0)),
            pl.BlockSpec((_H, 1), lambda i: (0, 0)),
        ],
        out_specs=[
            pl.BlockSpec((_RB, _H), lambda i: (i, 0)),
            pl.BlockSpec((_RB, 1), lambda i: (i, 0)),
        ],
        out_shape=[
            jax.ShapeDtypeStruct((_NPAD, _H), jnp.float32),
            jax.ShapeDtypeStruct((_NPAD, 1), jnp.float32),
        ],
    )(p, xcur, Wrel, Wroot, brel2d, pw2d)


def _select_body(k_sel, n_prev, *refs):
    s_ref, m_ref = refs[0], refs[1]
    prev_key_refs = refs[2:2 + n_prev]
    mo_ref, smo_ref, key_ref = refs[2 + n_prev:]

    score = s_ref[...]
    act = m_ref[...] > 0
    skey = _orderable_key(score)

    v = _kth_largest(skey, act, k_sel)
    sel = act & (skey > v)
    t = k_sel - _count(sel)
    tie = act & (skey == v)
    for pk_ref in prev_key_refs:
        pk = pk_ref[...]
        u = _kth_largest(pk, tie, t)
        g2 = tie & (pk > u)
        sel = sel | g2
        t = t - _count(g2)
        tie = tie & (pk == u)
    idx = (lax.broadcasted_iota(jnp.int32, (_SR, 128), 0) * 128
           + lax.broadcasted_iota(jnp.int32, (_SR, 128), 1))
    cut = _tth_smallest_nonneg(idx, tie, t, 14)
    sel = sel | (tie & (idx <= cut))
    mnew = sel.astype(jnp.float32)

    mo_ref[...] = mnew
    smo_ref[...] = score * mnew
    key_ref[...] = skey


def _select_tc(s80, m80, prev_keys, k_sel):
    return pl.pallas_call(
        functools.partial(_select_body, k_sel, len(prev_keys)),
        out_shape=[
            jax.ShapeDtypeStruct((_SR, 128), jnp.float32),
            jax.ShapeDtypeStruct((_SR, 128), jnp.float32),
            jax.ShapeDtypeStruct((_SR, 128), jnp.int32),
        ],
    )(s80, m80, *prev_keys)


def _scale_body(h_ref, sm_ref, mc_ref, xo_ref, g_ref):
    i = pl.program_id(0)
    xo = h_ref[...] * sm_ref[...]
    xo_ref[...] = xo
    bsum = jnp.sum(xo, axis=0, keepdims=True)
    bmax = jnp.max(jnp.where(mc_ref[...] > 0, xo, -jnp.inf),
                   axis=0, keepdims=True)
    cur = jnp.concatenate([bsum, bmax], axis=1)

    @pl.when(i == 0)
    def _():
        g_ref[...] = jnp.broadcast_to(cur, (8, 2 * _H))

    @pl.when(i != 0)
    def _():
        prev = g_ref[0:1, :]
        comb = jnp.concatenate(
            [prev[:, :_H] + bsum, jnp.maximum(prev[:, _H:], bmax)], axis=1)
        g_ref[...] = jnp.broadcast_to(comb, (8, 2 * _H))


def _scale_tc(h, sm_col, m_col):
    return pl.pallas_call(
        _scale_body,
        grid=(_NB,),
        in_specs=[
            pl.BlockSpec((_RB, _H), lambda i: (i, 0)),
            pl.BlockSpec((_RB, 1), lambda i: (i, 0)),
            pl.BlockSpec((_RB, 1), lambda i: (i, 0)),
        ],
        out_specs=[
            pl.BlockSpec((_RB, _H), lambda i: (i, 0)),
            pl.BlockSpec((8, 2 * _H), lambda i: (0, 0)),
        ],
        out_shape=[
            jax.ShapeDtypeStruct((_NPAD, _H), jnp.float32),
            jax.ShapeDtypeStruct((8, 2 * _H), jnp.float32),
        ],
    )(h, sm_col, m_col)


def _final_body(g0_ref, g1_ref, g2_ref, w1_ref, b1_ref, w2_ref, b2_ref, o_ref):
    gs = [g0_ref[0:1, :], g1_ref[0:1, :], g2_ref[0:1, :]]
    gmean = sum(g[:, :_H] / _KS[i] for i, g in enumerate(gs))
    gmax = gs[0][:, _H:] + gs[1][:, _H:] + gs[2][:, _H:]
    g = jnp.concatenate([gmean, gmax], axis=1)
    h = jnp.maximum(jnp.dot(g, w1_ref[...]) + b1_ref[...], 0.0)
    o = jnp.dot(h, w2_ref[...]) + b2_ref[...]
    o_ref[...] = jnp.broadcast_to(o, (8, _H))


def _final_tc(g0, g1, g2, W1, b1, W2p, b2p):
    return pl.pallas_call(
        _final_body,
        out_shape=jax.ShapeDtypeStruct((8, _H), jnp.float32),
    )(g0, g1, g2, W1, b1, W2p, b2p)


def kernel(x, edge_index, batch,
           Wrel0, brel0, Wroot0, pw0,
           Wrel1, brel1, Wroot1, pw1,
           Wrel2, brel2, Wroot2, pw2,
           lin1_W, lin1_b, lin2_W, lin2_b):
    f32 = jnp.float32
    src = edge_index[0].astype(jnp.int32)
    dst = edge_index[1].astype(jnp.int32)
    pad = _N + (jnp.arange(_EPAD - _E, dtype=jnp.int32) % (_NPAD - _N))
    src2d = jnp.concatenate([src, pad]).reshape(_EPAD // _C, _C)
    dst2d = jnp.concatenate([dst, pad]).reshape(_EPAD // _C, _C)
    xp = jnp.zeros((_NPAD, _H), f32).at[:_N].set(x.astype(f32))
    zrow = jnp.zeros((_ZR, _H), f32)
    m80 = (jnp.arange(_NPAD, dtype=jnp.int32) < _N).astype(f32).reshape(_SR, 128)
    W2p = jnp.zeros((_H, _H), f32).at[:, :_OUT].set(lin2_W)
    b2p = jnp.zeros((1, _H), f32).at[0, :_OUT].set(lin2_b)

    params = ((Wrel0, brel0, Wroot0, pw0),
              (Wrel1, brel1, Wroot1, pw1),
              (Wrel2, brel2, Wroot2, pw2))
    gs = []
    prev_keys = []
    for i in range(3):
        Wrel, brel, Wroot, pw = params[i]
        p = _seg_sum_sc(xp, src2d, dst2d, zrow)
        h, s_col = _conv_tc(p, xp, Wrel, Wroot,
                            brel.reshape(1, _H), pw.reshape(_H, 1))
        m80, sm80, key80 = _select_tc(s_col.reshape(_SR, 128), m80,
                                      prev_keys, _KS[i])
        xp, g = _scale_tc(h, sm80.reshape(_NPAD, 1), m80.reshape(_NPAD, 1))
        gs.append(g)
        prev_keys.insert(0, key80)
    o8 = _final_tc(gs[0], gs[1], gs[2], lin1_W, lin1_b.reshape(1, _H),
                   W2p, b2p)
    return (o8[0:1, :_OUT], jnp.zeros(()))


# speedup vs baseline: 1.0626x; 1.0000x over previous
import functools

import jax
import jax.numpy as jnp
from jax import lax
from jax.experimental import pallas as pl
from jax.experimental.pallas import tpu as pltpu
from jax.experimental.pallas import tpu_sc as plsc

_N = 10000
_E = 320000
_H = 128
_OUT = 10
_NPAD = 10240
_NC = 2
_NS = 16
_NW = _NC * _NS
_C = 64
_CPT = 160
_CPH = _CPT // 4
_EPAD = _C * _CPT * _NW
_ROWS_PER_TILE = _NPAD // _NS
_KS = (5000, 2500, 1250)
_INT_MIN_PY = -(2 ** 31)


_NSLOT = 4
_ZR = 32


def _seg_sum_body(x_hbm, src_hbm, dst_hbm, zrow_hbm, out_hbm,
                  sidx, didx, rows0, rows1, rows2, rows3, zv, agg,
                  gsem, ssem, zsem):
    c = lax.axis_index("c")
    s = lax.axis_index("s")
    w = c * _NS + s
    rows = (rows0, rows1, rows2, rows3)
    nzero = _ROWS_PER_TILE // _ZR

    pltpu.sync_copy(zrow_hbm, zv)
    for i in range(nzero):
        pltpu.async_copy(zv, agg.at[pl.ds(s * _ROWS_PER_TILE + i * _ZR, _ZR)],
                         zsem)

    for h in range(_CPT // _CPH):
        pltpu.sync_copy(src_hbm.at[pl.ds(w * _CPT + h * _CPH, _CPH)], sidx)
        pltpu.sync_copy(dst_hbm.at[pl.ds(w * _CPT + h * _CPH, _CPH)], didx)
        for b in range(_NSLOT):
            pltpu.async_copy(x_hbm.at[sidx.at[b]], rows[b], gsem[b])
        if h == 0:
            for i in range(nzero):
                pltpu.make_async_copy(
                    zv, agg.at[pl.ds(s * _ROWS_PER_TILE + i * _ZR, _ZR)],
                    zsem).wait()
            plsc.subcore_barrier()

        def ebody(jj, carry):
            for b in range(_NSLOT):
                cur = jj * _NSLOT + b
                pltpu.make_async_copy(x_hbm.at[sidx.at[cur]], rows[b],
                                      gsem[b]).wait()
                pltpu.async_copy(rows[b], agg.at[didx.at[cur]], ssem[b],
                                 add=True)
                nxt = cur + _NSLOT

                @pl.when(nxt < _CPH)
                def _():
                    pltpu.make_async_copy(rows[b], agg.at[didx.at[cur]],
                                          ssem[b]).wait()
                    pltpu.async_copy(x_hbm.at[sidx.at[nxt]], rows[b], gsem[b])
            return carry

        lax.fori_loop(0, _CPH // _NSLOT, ebody, 0)
        for b in range(_NSLOT):
            pltpu.make_async_copy(rows[b], agg.at[didx.at[0]], ssem[b]).wait()
    plsc.subcore_barrier()

    pltpu.sync_copy(agg.at[pl.ds(s * _ROWS_PER_TILE, _ROWS_PER_TILE)],
                    out_hbm.at[c, pl.ds(s * _ROWS_PER_TILE, _ROWS_PER_TILE)])


def _seg_sum_sc(x_pad, src2d, dst2d, zrow):
    mesh = plsc.VectorSubcoreMesh(core_axis_name="c", subcore_axis_name="s",
                                  num_cores=_NC, num_subcores=_NS)
    return pl.kernel(
        _seg_sum_body,
        out_type=jax.ShapeDtypeStruct((_NC, _NPAD, _H), jnp.float32),
        mesh=mesh,
        scratch_types=[
            pltpu.VMEM((_CPH, _C), jnp.int32),
            pltpu.VMEM((_CPH, _C), jnp.int32),
            pltpu.VMEM((_C, _H), jnp.float32),
            pltpu.VMEM((_C, _H), jnp.float32),
            pltpu.VMEM((_C, _H), jnp.float32),
            pltpu.VMEM((_C, _H), jnp.float32),
            pltpu.VMEM((_ZR, _H), jnp.float32),
            pltpu.VMEM_SHARED((_NPAD, _H), jnp.float32),
            [pltpu.SemaphoreType.DMA] * _NSLOT,
            [pltpu.SemaphoreType.DMA] * _NSLOT,
            pltpu.SemaphoreType.DMA,
        ],
    )(x_pad, src2d, dst2d, zrow)


def _orderable_key(score):
    raw = lax.bitcast_convert_type(score, jnp.int32)
    return jnp.where(raw >= 0, raw, jnp.int32(_INT_MIN_PY) - raw)


def _count(pred):
    return jnp.sum(pred.astype(jnp.int32))


def _kth_largest(keys, mask, t):
    int_min = jnp.int32(_INT_MIN_PY)
    c_pos = _count(mask & (keys >= 0))
    use_pos = c_pos >= t
    tt = jnp.where(use_pos, t, t - c_pos)
    tkeys = jnp.where(use_pos, keys, keys ^ int_min)

    def body(i, prefix):
        cand = prefix | lax.shift_left(jnp.int32(1), 30 - i)
        cnt = _count(mask & (tkeys >= cand))
        return jnp.where(cnt >= tt, cand, prefix)

    v = lax.fori_loop(0, 31, body, jnp.int32(0))
    return jnp.where(use_pos, v, v ^ int_min)


def _tth_smallest_nonneg(keys, mask, t, nbits):
    def body(i, prefix):
        cand = prefix | lax.shift_left(jnp.int32(1), nbits - 1 - i)
        cnt = _count(mask & (keys < cand))
        return jnp.where(cnt >= t, prefix, cand)

    return lax.fori_loop(0, nbits, body, jnp.int32(0))


_RB = 1024
_NB = _NPAD // _RB
_SR = _NPAD // 128


def _conv_body(p_ref, x_ref, wrel_ref, wroot_ref, brel_ref, pw_ref,
               h_ref, s_ref):
    agg = p_ref[0] + p_ref[1]
    h = jnp.maximum(
        jnp.dot(agg, wrel_ref[...]) + brel_ref[...]
        + jnp.dot(x_ref[...], wroot_ref[...]), 0.0)
    h_ref[...] = h
    pw = pw_ref[...]
    s_ref[...] = jnp.tanh(jnp.dot(h, pw) / jnp.sqrt(jnp.sum(pw * pw)))


def _conv_tc(p, xcur, Wrel, Wroot, brel2d, pw2d):
    return pl.pallas_call(
        _conv_body,
        grid=(_NB,),
        in_specs=[
            pl.BlockSpec((2, _RB, _H), lambda i: (0, i, 0)),
            pl.BlockSpec((_RB, _H), lambda i: (i, 0)),
            pl.BlockSpec((_H, _H), lambda i: (0, 0)),
            pl.BlockSpec((_H, _H), lambda i: (0, 0)),
            pl.BlockSpec((1, _H), lambda i: (0, 0)),
            pl.BlockSpec((_H, 1), lambda i: (0, 0)),
        ],
        out_specs=[
            pl.BlockSpec((_RB, _H), lambda i: (i, 0)),
            pl.BlockSpec((_RB, 1), lambda i: (i, 0)),
        ],
        out_shape=[
            jax.ShapeDtypeStruct((_NPAD, _H), jnp.float32),
            jax.ShapeDtypeStruct((_NPAD, 1), jnp.float32),
        ],
    )(p, xcur, Wrel, Wroot, brel2d, pw2d)


def _select_body(k_sel, n_prev, *refs):
    s_ref, m_ref = refs[0], refs[1]
    prev_key_refs = refs[2:2 + n_prev]
    mo_ref, smo_ref, key_ref = refs[2 + n_prev:]

    score = s_ref[...]
    act = m_ref[...] > 0
    skey = _orderable_key(score)

    v = _kth_largest(skey, act, k_sel)
    sel = act & (skey > v)
    t = k_sel - _count(sel)
    tie = act & (skey == v)
    for pk_ref in prev_key_refs:
        pk = pk_ref[...]
        u = _kth_largest(pk, tie, t)
        g2 = tie & (pk > u)
        sel = sel | g2
        t = t - _count(g2)
        tie = tie & (pk == u)
    idx = (lax.broadcasted_iota(jnp.int32, (_SR, 128), 0) * 128
           + lax.broadcasted_iota(jnp.int32, (_SR, 128), 1))
    cut = _tth_smallest_nonneg(idx, tie, t, 14)
    sel = sel | (tie & (idx <= cut))
    mnew = sel.astype(jnp.float32)

    mo_ref[...] = mnew
    smo_ref[...] = score * mnew
    key_ref[...] = skey


def _select_tc(s80, m80, prev_keys, k_sel):
    return pl.pallas_call(
        functools.partial(_select_body, k_sel, len(prev_keys)),
        out_shape=[
            jax.ShapeDtypeStruct((_SR, 128), jnp.float32),
            jax.ShapeDtypeStruct((_SR, 128), jnp.float32),
            jax.ShapeDtypeStruct((_SR, 128), jnp.int32),
        ],
    )(s80, m80, *prev_keys)


def _scale_body(h_ref, sm_ref, mc_ref, xo_ref, g_ref):
    i = pl.program_id(0)
    xo = h_ref[...] * sm_ref[...]
    xo_ref[...] = xo
    bsum = jnp.sum(xo, axis=0, keepdims=True)
    bmax = jnp.max(jnp.where(mc_ref[...] > 0, xo, -jnp.inf),
                   axis=0, keepdims=True)
    cur = jnp.concatenate([bsum, bmax], axis=1)

    @pl.when(i == 0)
    def _():
        g_ref[...] = jnp.broadcast_to(cur, (8, 2 * _H))

    @pl.when(i != 0)
    def _():
        prev = g_ref[0:1, :]
        comb = jnp.concatenate(
            [prev[:, :_H] + bsum, jnp.maximum(prev[:, _H:], bmax)], axis=1)
        g_ref[...] = jnp.broadcast_to(comb, (8, 2 * _H))


def _scale_tc(h, sm_col, m_col):
    return pl.pallas_call(
        _scale_body,
        grid=(_NB,),
        in_specs=[
            pl.BlockSpec((_RB, _H), lambda i: (i, 0)),
            pl.BlockSpec((_RB, 1), lambda i: (i, 0)),
            pl.BlockSpec((_RB, 1), lambda i: (i, 0)),
        ],
        out_specs=[
            pl.BlockSpec((_RB, _H), lambda i: (i, 0)),
            pl.BlockSpec((8, 2 * _H), lambda i: (0, 0)),
        ],
        out_shape=[
            jax.ShapeDtypeStruct((_NPAD, _H), jnp.float32),
            jax.ShapeDtypeStruct((8, 2 * _H), jnp.float32),
        ],
    )(h, sm_col, m_col)


def _final_body(g0_ref, g1_ref, g2_ref, w1_ref, b1_ref, w2_ref, b2_ref, o_ref):
    gs = [g0_ref[0:1, :], g1_ref[0:1, :], g2_ref[0:1, :]]
    gmean = sum(g[:, :_H] / _KS[i] for i, g in enumerate(gs))
    gmax = gs[0][:, _H:] + gs[1][:, _H:] + gs[2][:, _H:]
    g = jnp.concatenate([gmean, gmax], axis=1)
    h = jnp.maximum(jnp.dot(g, w1_ref[...]) + b1_ref[...], 0.0)
    o = jnp.dot(h, w2_ref[...]) + b2_ref[...]
    o_ref[...] = jnp.broadcast_to(o, (8, _H))


def _final_tc(g0, g1, g2, W1, b1, W2p, b2p):
    return pl.pallas_call(
        _final_body,
        out_shape=jax.ShapeDtypeStruct((8, _H), jnp.float32),
    )(g0, g1, g2, W1, b1, W2p, b2p)


def kernel(x, edge_index, batch,
           Wrel0, brel0, Wroot0, pw0,
           Wrel1, brel1, Wroot1, pw1,
           Wrel2, brel2, Wroot2, pw2,
           lin1_W, lin1_b, lin2_W, lin2_b):
    f32 = jnp.float32
    src = edge_index[0].astype(jnp.int32)
    dst = edge_index[1].astype(jnp.int32)
    pad = _N + (jnp.arange(_EPAD - _E, dtype=jnp.int32) % (_NPAD - _N))
    src2d = jnp.concatenate([src, pad]).reshape(_EPAD // _C, _C)
    dst2d = jnp.concatenate([dst, pad]).reshape(_EPAD // _C, _C)
    xp = jnp.zeros((_NPAD, _H), f32).at[:_N].set(x.astype(f32))
    zrow = jnp.zeros((_ZR, _H), f32)
    m80 = (jnp.arange(_NPAD, dtype=jnp.int32) < _N).astype(f32).reshape(_SR, 128)
    W2p = jnp.zeros((_H, _H), f32).at[:, :_OUT].set(lin2_W)
    b2p = jnp.zeros((1, _H), f32).at[0, :_OUT].set(lin2_b)

    params = ((Wrel0, brel0, Wroot0, pw0),
              (Wrel1, brel1, Wroot1, pw1),
              (Wrel2, brel2, Wroot2, pw2))
    gs = []
    prev_keys = []
    for i in range(3):
        Wrel, brel, Wroot, pw = params[i]
        p = _seg_sum_sc(xp, src2d, dst2d, zrow)
        h, s_col = _conv_tc(p, xp, Wrel, Wroot,
                            brel.reshape(1, _H), pw.reshape(_H, 1))
        m80, sm80, key80 = _select_tc(s_col.reshape(_SR, 128), m80,
                                      prev_keys, _KS[i])
        xp, g = _scale_tc(h, sm80.reshape(_NPAD, 1), m80.reshape(_NPAD, 1))
        gs.append(g)
        prev_keys.insert(0, key80)
    o8 = _final_tc(gs[0], gs[1], gs[2], lin1_W, lin1_b.reshape(1, _H),
                   W2p, b2p)
    return (o8[0:1, :_OUT], jnp.zeros(()))
